# baseline (speedup 1.0000x reference)
import jax
import jax.numpy as jnp
from jax.experimental import pallas as pl
from jax.experimental.pallas import tpu as pltpu

N_ROWS = 200000
STATES = 128
S2 = 2 * STATES
SIT = 64
CH = 10000
NCH = N_ROWS // CH
NBUF = 6
LOOK = 3
GBLK = 8


def _body(idx_ref, x_ref, wzt_ref, wrt_ref, wht_ref, uzt_ref, urt_ref,
          uht_ref, bz_ref, br_ref, bh_ref, gb_ref, gp_ref, st_ref,
          out_ref, bufs, rowb_ref, rowp_ref, sin, sout, wsem):
    def in_cp(g):
        return pltpu.make_async_copy(
            st_ref.at[pl.ds(g * CH, CH), :], bufs.at[g % NBUF],
            sin.at[g % NBUF])

    def out_cp(g):
        return pltpu.make_async_copy(
            bufs.at[g % NBUF], out_ref.at[pl.ds(g * CH, CH), :],
            sout.at[g % NBUF])

    for g in range(LOOK):
        in_cp(g).start()

    rb = idx_ref[0] % GBLK
    rp = idx_ref[1] % GBLK
    h_b = gb_ref[pl.ds(rb, 1), :]
    h_p = gp_ref[pl.ds(rp, 1), :]
    h = jnp.concatenate([h_b, h_p], axis=1)
    xv = x_ref[...]
    hi = jax.lax.Precision.HIGHEST
    wx_z = jax.lax.dot(xv, wzt_ref[...], precision=hi)
    wx_r = jax.lax.dot(xv, wrt_ref[...], precision=hi)
    wx_h = jax.lax.dot(xv, wht_ref[...], precision=hi)
    z = jax.nn.sigmoid(wx_z + jax.lax.dot(h, uzt_ref[...], precision=hi)
                       + bz_ref[...])
    r = jax.nn.sigmoid(wx_r + jax.lax.dot(h, urt_ref[...], precision=hi)
                       - br_ref[...])
    m = jnp.tanh(wx_h + jax.lax.dot(r * h, uht_ref[...], precision=hi)
                 + bh_ref[...])
    hp_new = z * h + (1.0 - z) * m
    rowb_ref[...] = hp_new[:, :STATES]
    rowp_ref[...] = hp_new[:, STATES:]

    for g in range(NCH):
        nxt = g + LOOK
        if nxt < NCH:
            if nxt >= NBUF:
                out_cp(nxt - NBUF).wait()
            in_cp(nxt).start()
        in_cp(g).wait()
        out_cp(g).start()

    for g in range(NCH - NBUF, NCH):
        out_cp(g).wait()

    wb = pltpu.make_async_copy(
        rowb_ref, out_ref.at[pl.ds(idx_ref[0], 1), :], wsem)
    wb.start()
    wb.wait()
    wp = pltpu.make_async_copy(
        rowp_ref, out_ref.at[pl.ds(idx_ref[1], 1), :], wsem)
    wp.start()
    wp.wait()


def kernel(x, b, p, state, Wz, Wr, Wh, Uz, Ur, Uh, bz, br, bh):
    st = state.reshape(N_ROWS, STATES)
    idx = jnp.concatenate([b, p]).astype(jnp.int32)
    full = lambda shape: pl.BlockSpec(shape, lambda i, s: (0,) * len(shape))
    grid_spec = pltpu.PrefetchScalarGridSpec(
        num_scalar_prefetch=1,
        grid=(1,),
        in_specs=[
            full((1, SIT)),
            full((SIT, S2)),
            full((SIT, S2)),
            full((SIT, S2)),
            full((S2, S2)),
            full((S2, S2)),
            full((S2, S2)),
            full((1, S2)),
            full((1, S2)),
            full((1, S2)),
            pl.BlockSpec((GBLK, STATES), lambda i, s: (s[0] // GBLK, 0)),
            pl.BlockSpec((GBLK, STATES), lambda i, s: (s[1] // GBLK, 0)),
            pl.BlockSpec(memory_space=pl.ANY),
        ],
        out_specs=pl.BlockSpec(memory_space=pl.ANY),
        scratch_shapes=[
            pltpu.VMEM((NBUF, CH, STATES), jnp.float32),
            pltpu.VMEM((1, STATES), jnp.float32),
            pltpu.VMEM((1, STATES), jnp.float32),
            pltpu.SemaphoreType.DMA((NBUF,)),
            pltpu.SemaphoreType.DMA((NBUF,)),
            pltpu.SemaphoreType.DMA,
        ],
    )
    out = pl.pallas_call(
        _body,
        grid_spec=grid_spec,
        out_shape=jax.ShapeDtypeStruct((N_ROWS, STATES), jnp.float32),
    )(idx, x.reshape(1, SIT), Wz.T, Wr.T, Wh.T, Uz.T, Ur.T, Uh.T,
      bz.reshape(1, S2), br.reshape(1, S2), bh.reshape(1, S2),
      st, st, st)
    return out.reshape(1, N_ROWS, STATES)


# speedup vs baseline: 1.0271x; 1.0271x over previous
import jax
import jax.numpy as jnp
from jax.experimental import pallas as pl
from jax.experimental.pallas import tpu as pltpu

N_ROWS = 200000
STATES = 128
S2 = 2 * STATES
SIT = 64
CHUNKS = [2400, 4800, 9600] + [20800] * 8 + [9600, 4800, 2400]
OFFS = [sum(CHUNKS[:i]) for i in range(len(CHUNKS))]
NCH = len(CHUNKS)
CHMAX = max(CHUNKS)
NBUF = 4
LOOK = 2
GBLK = 8


def _body(idx_ref, x_ref, wzt_ref, wrt_ref, wht_ref, uzt_ref, urt_ref,
          uht_ref, bz_ref, br_ref, bh_ref, gb_ref, gp_ref, st_ref,
          out_ref, bufs, rowb_ref, rowp_ref, sin, sout, wsem, psem):
    def in_cp(g):
        return pltpu.make_async_copy(
            st_ref.at[pl.ds(OFFS[g], CHUNKS[g]), :],
            bufs.at[g % NBUF, pl.ds(0, CHUNKS[g]), :],
            sin.at[g % NBUF])

    def out_cp(g):
        return pltpu.make_async_copy(
            bufs.at[g % NBUF, pl.ds(0, CHUNKS[g]), :],
            out_ref.at[pl.ds(OFFS[g], CHUNKS[g]), :],
            sout.at[g % NBUF])

    def row_writes_for(g):
        lo, hi = OFFS[g], OFFS[g] + CHUNKS[g]

        @pl.when(jnp.logical_and(idx_ref[0] >= lo, idx_ref[0] < hi))
        def _():
            pltpu.make_async_copy(
                rowb_ref, out_ref.at[pl.ds(idx_ref[0], 1), :], wsem).start()

        @pl.when(jnp.logical_and(idx_ref[1] >= lo, idx_ref[1] < hi))
        def _():
            pltpu.make_async_copy(
                rowp_ref, out_ref.at[pl.ds(idx_ref[1], 1), :], psem).start()

    for g in range(LOOK):
        in_cp(g).start()

    rb = idx_ref[0] % GBLK
    rp = idx_ref[1] % GBLK
    h_b = gb_ref[pl.ds(rb, 1), :]
    h_p = gp_ref[pl.ds(rp, 1), :]
    h = jnp.concatenate([h_b, h_p], axis=1)
    xv = x_ref[...]
    hi = jax.lax.Precision.HIGHEST
    wx_z = jax.lax.dot(xv, wzt_ref[...], precision=hi)
    wx_r = jax.lax.dot(xv, wrt_ref[...], precision=hi)
    wx_h = jax.lax.dot(xv, wht_ref[...], precision=hi)
    z = jax.nn.sigmoid(wx_z + jax.lax.dot(h, uzt_ref[...], precision=hi)
                       + bz_ref[...])
    r = jax.nn.sigmoid(wx_r + jax.lax.dot(h, urt_ref[...], precision=hi)
                       - br_ref[...])
    m = jnp.tanh(wx_h + jax.lax.dot(r * h, uht_ref[...], precision=hi)
                 + bh_ref[...])
    hp_new = z * h + (1.0 - z) * m
    rowb_ref[...] = hp_new[:, :STATES]
    rowp_ref[...] = hp_new[:, STATES:]

    for g in range(NCH):
        nxt = g + LOOK
        if nxt < NCH:
            if nxt >= NBUF:
                out_cp(nxt - NBUF).wait()
                row_writes_for(nxt - NBUF)
            in_cp(nxt).start()
        in_cp(g).wait()
        out_cp(g).start()

    for g in range(NCH - NBUF, NCH):
        out_cp(g).wait()
        row_writes_for(g)

    pltpu.make_async_copy(
        rowb_ref, out_ref.at[pl.ds(idx_ref[0], 1), :], wsem).wait()
    pltpu.make_async_copy(
        rowp_ref, out_ref.at[pl.ds(idx_ref[1], 1), :], psem).wait()


def kernel(x, b, p, state, Wz, Wr, Wh, Uz, Ur, Uh, bz, br, bh):
    st = state.reshape(N_ROWS, STATES)
    idx = jnp.concatenate([b, p]).astype(jnp.int32)
    full = lambda shape: pl.BlockSpec(shape, lambda i, s: (0,) * len(shape))
    grid_spec = pltpu.PrefetchScalarGridSpec(
        num_scalar_prefetch=1,
        grid=(1,),
        in_specs=[
            full((1, SIT)),
            full((SIT, S2)),
            full((SIT, S2)),
            full((SIT, S2)),
            full((S2, S2)),
            full((S2, S2)),
            full((S2, S2)),
            full((1, S2)),
            full((1, S2)),
            full((1, S2)),
            pl.BlockSpec((GBLK, STATES), lambda i, s: (s[0] // GBLK, 0)),
            pl.BlockSpec((GBLK, STATES), lambda i, s: (s[1] // GBLK, 0)),
            pl.BlockSpec(memory_space=pl.ANY),
        ],
        out_specs=pl.BlockSpec(memory_space=pl.ANY),
        scratch_shapes=[
            pltpu.VMEM((NBUF, CHMAX, STATES), jnp.float32),
            pltpu.VMEM((1, STATES), jnp.float32),
            pltpu.VMEM((1, STATES), jnp.float32),
            pltpu.SemaphoreType.DMA((NBUF,)),
            pltpu.SemaphoreType.DMA((NBUF,)),
            pltpu.SemaphoreType.DMA,
            pltpu.SemaphoreType.DMA,
        ],
    )
    out = pl.pallas_call(
        _body,
        grid_spec=grid_spec,
        out_shape=jax.ShapeDtypeStruct((N_ROWS, STATES), jnp.float32),
    )(idx, x.reshape(1, SIT), Wz.T, Wr.T, Wh.T, Uz.T, Ur.T, Uh.T,
      bz.reshape(1, S2), br.reshape(1, S2), bh.reshape(1, S2),
      st, st, st)
    return out.reshape(1, N_ROWS, STATES)


# speedup vs baseline: 1.0424x; 1.0148x over previous
import jax
import jax.numpy as jnp
from jax.experimental import pallas as pl
from jax.experimental.pallas import tpu as pltpu

N_ROWS = 200000
STATES = 128
S2 = 2 * STATES
SIT = 64
BLK = 25000
NBLK = N_ROWS // BLK
GBLK = 8


def _body(idx_ref, x_ref, wzt_ref, wrt_ref, wht_ref, uzt_ref, urt_ref,
          uht_ref, bz_ref, br_ref, bh_ref, gb_ref, gp_ref, st_ref,
          out_ref, dh_ref):
    i = pl.program_id(0)

    @pl.when(i == 0)
    def _compute_gru():
        rb = idx_ref[0] % GBLK
        rp = idx_ref[1] % GBLK
        h_b = gb_ref[pl.ds(rb, 1), :]
        h_p = gp_ref[pl.ds(rp, 1), :]
        h = jnp.concatenate([h_b, h_p], axis=1)
        xv = x_ref[...]
        hi = jax.lax.Precision.HIGHEST
        wx_z = jax.lax.dot(xv, wzt_ref[...], precision=hi)
        wx_r = jax.lax.dot(xv, wrt_ref[...], precision=hi)
        wx_h = jax.lax.dot(xv, wht_ref[...], precision=hi)
        z = jax.nn.sigmoid(wx_z + jax.lax.dot(h, uzt_ref[...], precision=hi)
                           + bz_ref[...])
        r = jax.nn.sigmoid(wx_r + jax.lax.dot(h, urt_ref[...], precision=hi)
                           - br_ref[...])
        m = jnp.tanh(wx_h + jax.lax.dot(r * h, uht_ref[...], precision=hi)
                     + bh_ref[...])
        hp_new = z * h + (1.0 - z) * m
        dh_ref[...] = hp_new - h

    out_ref[...] = st_ref[...]

    row_b = idx_ref[0]
    row_p = idx_ref[1]
    lo = i * BLK

    @pl.when(jnp.logical_and(row_b >= lo, row_b < lo + BLK))
    def _add_b():
        r = row_b - lo
        out_ref[pl.ds(r, 1), :] = out_ref[pl.ds(r, 1), :] + dh_ref[:, :STATES]

    @pl.when(jnp.logical_and(row_p >= lo, row_p < lo + BLK))
    def _add_p():
        r = row_p - lo
        out_ref[pl.ds(r, 1), :] = out_ref[pl.ds(r, 1), :] + dh_ref[:, STATES:]


def kernel(x, b, p, state, Wz, Wr, Wh, Uz, Ur, Uh, bz, br, bh):
    st = state.reshape(N_ROWS, STATES)
    idx = jnp.concatenate([b, p]).astype(jnp.int32)
    full = lambda arr: pl.BlockSpec(arr.shape, lambda i, s: (0,) * arr.ndim)
    grid_spec = pltpu.PrefetchScalarGridSpec(
        num_scalar_prefetch=1,
        grid=(NBLK,),
        in_specs=[
            full(jnp.zeros((1, SIT))),
            full(jnp.zeros((SIT, S2))),
            full(jnp.zeros((SIT, S2))),
            full(jnp.zeros((SIT, S2))),
            full(jnp.zeros((S2, S2))),
            full(jnp.zeros((S2, S2))),
            full(jnp.zeros((S2, S2))),
            full(jnp.zeros((1, S2))),
            full(jnp.zeros((1, S2))),
            full(jnp.zeros((1, S2))),
            pl.BlockSpec((GBLK, STATES), lambda i, s: (s[0] // GBLK, 0)),
            pl.BlockSpec((GBLK, STATES), lambda i, s: (s[1] // GBLK, 0)),
            pl.BlockSpec((BLK, STATES), lambda i, s: (i, 0)),
        ],
        out_specs=pl.BlockSpec((BLK, STATES), lambda i, s: (i, 0)),
        scratch_shapes=[pltpu.VMEM((1, S2), jnp.float32)],
    )
    out = pl.pallas_call(
        _body,
        grid_spec=grid_spec,
        out_shape=jax.ShapeDtypeStruct((N_ROWS, STATES), jnp.float32),
    )(idx, x.reshape(1, SIT), Wz.T, Wr.T, Wh.T, Uz.T, Ur.T, Uh.T,
      bz.reshape(1, S2), br.reshape(1, S2), bh.reshape(1, S2),
      st, st, st)
    return out.reshape(1, N_ROWS, STATES)
